# baseline (speedup 1.0000x reference)
import functools

import jax
import jax.numpy as jnp
from jax import lax
from jax.experimental import pallas as pl
from jax.experimental.pallas import tpu as pltpu
from jax.experimental.pallas import tpu_sc as plsc

_NUM_GROUPS = 2
_NUM_VARS = 320

_SC_CORES = 1
_SC_SUBCORES = 16
_SC_WORKERS = _SC_CORES * _SC_SUBCORES


def _argmax_first(h):
    v = h.shape[1]
    m = jnp.max(h, axis=1, keepdims=True)
    iota = lax.broadcasted_iota(jnp.int32, h.shape, 1)
    cand = jnp.where(h == m, iota, v)
    return jnp.min(cand, axis=1, keepdims=True)


def _tc_body(total_tokens, hs_ref, w_ref, b_ref, idx_ref, plex_ref,
             w0_ref, w1_ref, counts_ref):
    i = pl.program_id(0)

    @pl.when(i == 0)
    def _():
        w0_ref[...] = w_ref[:, :_NUM_VARS].astype(jnp.bfloat16)
        w1_ref[...] = w_ref[:, _NUM_VARS:].astype(jnp.bfloat16)
        counts_ref[...] = jnp.zeros_like(counts_ref)

    hs = hs_ref[...].astype(jnp.bfloat16)
    dot = functools.partial(
        lax.dot_general,
        dimension_numbers=(((1,), (0,)), ((), ())),
        preferred_element_type=jnp.float32,
    )
    h0 = dot(hs, w0_ref[...]) + b_ref[0:1, :_NUM_VARS]
    h1 = dot(hs, w1_ref[...]) + b_ref[0:1, _NUM_VARS:]

    idx0 = _argmax_first(h0)
    idx1 = _argmax_first(h1)
    idx_ref[...] = jnp.concatenate(
        [idx0.reshape(1, -1), idx1.reshape(1, -1) + _NUM_VARS], axis=0)

    iota = lax.broadcasted_iota(jnp.int32, h0.shape, 1)
    c0 = jnp.sum(jnp.where(iota == idx0, 1.0, 0.0), axis=0, keepdims=True)
    c1 = jnp.sum(jnp.where(iota == idx1, 1.0, 0.0), axis=0, keepdims=True)
    counts_ref[0:1, :] += c0
    counts_ref[1:2, :] += c1

    @pl.when(i == pl.num_programs(0) - 1)
    def _():
        p = counts_ref[...] * (1.0 / total_tokens)
        ent = jnp.sum(p * jnp.log(p + 1e-7), axis=1, keepdims=True)
        plex_ref[...] = jnp.sum(jnp.exp(-ent), axis=0, keepdims=True)


def _tc_quantize(hs2d, w2d, b2d, block_tokens=1024):
    t, hidden = hs2d.shape
    grid = t // block_tokens
    return pl.pallas_call(
        functools.partial(_tc_body, t),
        grid=(grid,),
        in_specs=[
            pl.BlockSpec((block_tokens, hidden), lambda i: (i, 0)),
            pl.BlockSpec(w2d.shape, lambda i: (0, 0)),
            pl.BlockSpec(b2d.shape, lambda i: (0, 0)),
        ],
        out_specs=[
            pl.BlockSpec((_NUM_GROUPS, block_tokens), lambda i: (0, i)),
            pl.BlockSpec((1, 1), lambda i: (0, 0)),
        ],
        out_shape=[
            jax.ShapeDtypeStruct((_NUM_GROUPS, t), jnp.int32),
            jax.ShapeDtypeStruct((1, 1), jnp.float32),
        ],
        scratch_shapes=[
            pltpu.VMEM((hidden, _NUM_VARS), jnp.bfloat16),
            pltpu.VMEM((hidden, _NUM_VARS), jnp.bfloat16),
            pltpu.VMEM((_NUM_GROUPS, _NUM_VARS), jnp.float32),
        ],
    )(hs2d, w2d, b2d)


def _sc_gather(table, idx, t):
    d = table.shape[1]
    w_per_g = _SC_WORKERS // _NUM_GROUPS
    tok_per_w = t // w_per_g
    mesh = plsc.VectorSubcoreMesh(core_axis_name="c", subcore_axis_name="s",
                                  num_cores=_SC_CORES)

    @functools.partial(
        pl.kernel,
        mesh=mesh,
        out_type=jax.ShapeDtypeStruct((t, _NUM_GROUPS * d), jnp.float32),
        scratch_types=[
            pltpu.VMEM((tok_per_w,), jnp.int32),
            pltpu.VMEM((tok_per_w, d), jnp.float32),
            pltpu.SemaphoreType.DMA,
        ],
    )
    def k(table_hbm, idx_hbm, out_hbm, idx_v, rows_v, sem):
        wid = lax.axis_index("s") * _SC_CORES + lax.axis_index("c")
        group = wid // w_per_g
        tok0 = (wid % w_per_g) * tok_per_w
        pltpu.sync_copy(idx_hbm.at[group, pl.ds(tok0, tok_per_w)], idx_v)
        pltpu.async_copy(table_hbm.at[idx_v], rows_v, sem).wait()
        pltpu.sync_copy(rows_v, out_hbm.at[pl.ds(tok0, tok_per_w),
                                           pl.ds(group * d, d)])

    return k(table, idx)


def kernel(hidden_states, W, b, codevectors):
    batch, seq, hidden = hidden_states.shape
    t = batch * seq
    hs2d = hidden_states.reshape(t, hidden)
    b2d = b.reshape(1, _NUM_GROUPS * _NUM_VARS)

    idx_gm, plex = _tc_quantize(hs2d, W, b2d)

    table = codevectors.reshape(codevectors.shape[1], codevectors.shape[2])
    cv = _sc_gather(table, idx_gm, t)
    return cv.reshape(batch, seq, _NUM_GROUPS * table.shape[1]), plex[0, 0]


# speedup vs baseline: 1.0104x; 1.0104x over previous
import functools

import jax
import jax.numpy as jnp
from jax import lax
from jax.experimental import pallas as pl
from jax.experimental.pallas import tpu as pltpu
from jax.experimental.pallas import tpu_sc as plsc

_NUM_GROUPS = 2
_NUM_VARS = 320

_SC_CORES = 1
_SC_SUBCORES = 16
_SC_WORKERS = _SC_CORES * _SC_SUBCORES


def _argmax_first(h):
    v = h.shape[1]
    m = jnp.max(h, axis=1, keepdims=True)
    iota = lax.broadcasted_iota(jnp.int32, h.shape, 1)
    cand = jnp.where(h == m, iota, v)
    return jnp.min(cand, axis=1, keepdims=True)


def _tc_body(total_tokens, hs_ref, w_ref, b_ref, idx_ref, plex_ref,
             w0_ref, w1_ref, counts_ref):
    i = pl.program_id(0)

    @pl.when(i == 0)
    def _():
        w0_ref[...] = w_ref[:, :_NUM_VARS].astype(jnp.bfloat16)
        w1_ref[...] = w_ref[:, _NUM_VARS:].astype(jnp.bfloat16)
        counts_ref[...] = jnp.zeros_like(counts_ref)

    hs = hs_ref[...].astype(jnp.bfloat16)
    dot = functools.partial(
        lax.dot_general,
        dimension_numbers=(((1,), (0,)), ((), ())),
        preferred_element_type=jnp.float32,
    )
    h0 = dot(hs, w0_ref[...]) + b_ref[0:1, :_NUM_VARS]
    h1 = dot(hs, w1_ref[...]) + b_ref[0:1, _NUM_VARS:]

    idx0 = _argmax_first(h0)
    idx1 = _argmax_first(h1)
    idx_ref[...] = jnp.concatenate(
        [idx0.reshape(1, -1), idx1.reshape(1, -1) + _NUM_VARS], axis=0)

    iota = lax.broadcasted_iota(jnp.int32, h0.shape, 1)
    c0 = jnp.sum(jnp.where(iota == idx0, 1.0, 0.0), axis=0, keepdims=True)
    c1 = jnp.sum(jnp.where(iota == idx1, 1.0, 0.0), axis=0, keepdims=True)
    counts_ref[0:1, :] += c0
    counts_ref[1:2, :] += c1

    @pl.when(i == pl.num_programs(0) - 1)
    def _():
        p = counts_ref[...] * (1.0 / total_tokens)
        ent = jnp.sum(p * jnp.log(p + 1e-7), axis=1, keepdims=True)
        plex_ref[...] = jnp.sum(jnp.exp(-ent), axis=0, keepdims=True)


def _tc_quantize(hs2d, w2d, b2d, block_tokens=512):
    t, hidden = hs2d.shape
    grid = t // block_tokens
    return pl.pallas_call(
        functools.partial(_tc_body, t),
        grid=(grid,),
        in_specs=[
            pl.BlockSpec((block_tokens, hidden), lambda i: (i, 0)),
            pl.BlockSpec(w2d.shape, lambda i: (0, 0)),
            pl.BlockSpec(b2d.shape, lambda i: (0, 0)),
        ],
        out_specs=[
            pl.BlockSpec((_NUM_GROUPS, block_tokens), lambda i: (0, i)),
            pl.BlockSpec((1, 1), lambda i: (0, 0)),
        ],
        out_shape=[
            jax.ShapeDtypeStruct((_NUM_GROUPS, t), jnp.int32),
            jax.ShapeDtypeStruct((1, 1), jnp.float32),
        ],
        scratch_shapes=[
            pltpu.VMEM((hidden, _NUM_VARS), jnp.bfloat16),
            pltpu.VMEM((hidden, _NUM_VARS), jnp.bfloat16),
            pltpu.VMEM((_NUM_GROUPS, _NUM_VARS), jnp.float32),
        ],
    )(hs2d, w2d, b2d)


def _sc_gather(table, idx, t):
    d = table.shape[1]
    w_per_g = _SC_WORKERS // _NUM_GROUPS
    tok_per_w = t // w_per_g
    mesh = plsc.VectorSubcoreMesh(core_axis_name="c", subcore_axis_name="s",
                                  num_cores=_SC_CORES)

    half = tok_per_w // 2

    @functools.partial(
        pl.kernel,
        mesh=mesh,
        out_type=jax.ShapeDtypeStruct((t, _NUM_GROUPS * d), jnp.float32),
        scratch_types=[
            pltpu.VMEM((half,), jnp.int32),
            pltpu.VMEM((half,), jnp.int32),
            pltpu.VMEM((half, d), jnp.float32),
            pltpu.VMEM((half, d), jnp.float32),
            pltpu.SemaphoreType.DMA,
            pltpu.SemaphoreType.DMA,
            pltpu.SemaphoreType.DMA,
            pltpu.SemaphoreType.DMA,
        ],
    )
    def k(table_hbm, idx_hbm, out_hbm, idx_a, idx_b, rows_a, rows_b,
          sa, sb, so_a, so_b):
        wid = lax.axis_index("s") * _SC_CORES + lax.axis_index("c")
        group = wid // w_per_g
        tok0 = (wid % w_per_g) * tok_per_w
        col = group * d
        pltpu.sync_copy(idx_hbm.at[group, pl.ds(tok0, half)], idx_a)
        ga = pltpu.async_copy(table_hbm.at[idx_a], rows_a, sa)
        pltpu.sync_copy(idx_hbm.at[group, pl.ds(tok0 + half, half)], idx_b)
        gb = pltpu.async_copy(table_hbm.at[idx_b], rows_b, sb)
        ga.wait()
        oa = pltpu.async_copy(
            rows_a, out_hbm.at[pl.ds(tok0, half), pl.ds(col, d)], so_a)
        gb.wait()
        ob = pltpu.async_copy(
            rows_b, out_hbm.at[pl.ds(tok0 + half, half), pl.ds(col, d)], so_b)
        oa.wait()
        ob.wait()

    return k(table, idx)


def kernel(hidden_states, W, b, codevectors):
    batch, seq, hidden = hidden_states.shape
    t = batch * seq
    hs2d = hidden_states.reshape(t, hidden)
    b2d = b.reshape(1, _NUM_GROUPS * _NUM_VARS)

    idx_gm, plex = _tc_quantize(hs2d, W, b2d)

    table = codevectors.reshape(codevectors.shape[1], codevectors.shape[2])
    cv = _sc_gather(table, idx_gm, t)
    return cv.reshape(batch, seq, _NUM_GROUPS * table.shape[1]), plex[0, 0]
